# initial kernel scaffold (unmeasured)
import jax
import jax.numpy as jnp
from jax import lax
from jax.experimental import pallas as pl
from jax.experimental.pallas import tpu as pltpu

N_DEV = 4
N_EXP = 32
E_LOC = 8
CAP = 204
N_TOK = 2048
D = 512
H = 1024


def _ring_ag_routes(route_idx):
    r16 = route_idx.reshape(16, 128)

    def body(r_ref, out_ref, comm, ssem, rsem):
        me = lax.axis_index("i")
        left = (me - 1) % N_DEV
        right = (me + 1) % N_DEV
        bar = pltpu.get_barrier_semaphore()
        for nbr in (left, right):
            pl.semaphore_signal(
                bar, inc=1, device_id=(nbr,),
                device_id_type=pl.DeviceIdType.MESH,
            )
        pl.semaphore_wait(bar, 2)

        out_ref[pl.ds(me * 16, 16), :] = r_ref[...]
        for h in range(N_DEV - 1):
            src = r_ref if h == 0 else comm.at[h - 1]
            rdma = pltpu.make_async_remote_copy(
                src_ref=src,
                dst_ref=comm.at[h],
                send_sem=ssem.at[h],
                recv_sem=rsem.at[h],
                device_id=(right,),
                device_id_type=pl.DeviceIdType.MESH,
            )
            rdma.start()
            rdma.wait()
            origin = (me - 1 - h) % N_DEV
            out_ref[pl.ds(origin * 16, 16), :] = comm[h]

    out = pl.pallas_call(
        body,
        out_shape=jax.ShapeDtypeStruct((N_DEV * 16, 128), jnp.int32),
        in_specs=[pl.BlockSpec(memory_space=pltpu.VMEM)],
        out_specs=pl.BlockSpec(memory_space=pltpu.VMEM),
        scratch_shapes=[
            pltpu.VMEM((N_DEV - 1, 16, 128), jnp.int32),
            pltpu.SemaphoreType.DMA((N_DEV - 1,)),
            pltpu.SemaphoreType.DMA((N_DEV - 1,)),
        ],
        compiler_params=pltpu.CompilerParams(collective_id=0),
    )(r16)
    return out.reshape(N_DEV * N_TOK)


def _moe_ring(x_bf, mask_bf, w_bf):

    def body(x_ref, m_ref, w_ref, out_ref, comm, ssem, rsem):
        me = lax.axis_index("i")
        left = (me - 1) % N_DEV
        right = (me + 1) % N_DEV
        bar = pltpu.get_barrier_semaphore()
        for nbr in (left, right):
            pl.semaphore_signal(
                bar, inc=1, device_id=(nbr,),
                device_id_type=pl.DeviceIdType.MESH,
            )
        pl.semaphore_wait(bar, 2)

        xv = x_ref[...]
        mv = m_ref[...]

        def block(get_w, h):
            acc = None
            for k in range(E_LOC):
                col = h * E_LOC + k
                xm = xv * mv[:, col:col + 1]
                p = jnp.dot(xm, get_w(k), preferred_element_type=jnp.float32)
                acc = p if acc is None else acc + p
            return acc

        r0 = pltpu.make_async_remote_copy(
            src_ref=w_ref,
            dst_ref=comm.at[0],
            send_sem=ssem.at[0],
            recv_sem=rsem.at[0],
            device_id=(right,),
            device_id_type=pl.DeviceIdType.MESH,
        )
        r0.start()
        out_ref[...] = block(lambda k: w_ref[k], 0)
        r0.wait()

        for h in range(1, N_DEV - 1):
            r = pltpu.make_async_remote_copy(
                src_ref=comm.at[h - 1],
                dst_ref=comm.at[h],
                send_sem=ssem.at[h],
                recv_sem=rsem.at[h],
                device_id=(right,),
                device_id_type=pl.DeviceIdType.MESH,
            )
            r.start()
            out_ref[...] += block(lambda k: comm[h - 1, k], h)
            r.wait()

        out_ref[...] += block(lambda k: comm[N_DEV - 2, k], N_DEV - 1)

    return pl.pallas_call(
        body,
        out_shape=jax.ShapeDtypeStruct((N_TOK, H), jnp.float32),
        in_specs=[
            pl.BlockSpec(memory_space=pltpu.VMEM),
            pl.BlockSpec(memory_space=pltpu.VMEM),
            pl.BlockSpec(memory_space=pltpu.VMEM),
        ],
        out_specs=pl.BlockSpec(memory_space=pltpu.VMEM),
        scratch_shapes=[
            pltpu.VMEM((N_DEV - 1, E_LOC, D, H), jnp.bfloat16),
            pltpu.SemaphoreType.DMA((N_DEV - 1,)),
            pltpu.SemaphoreType.DMA((N_DEV - 1,)),
        ],
        compiler_params=pltpu.CompilerParams(collective_id=1),
    )(x_bf, mask_bf, w_bf)


def kernel(x, router_W, route_idx, expert_W):
    del router_W

    me = lax.axis_index("i")

    e_g = _ring_ag_routes(route_idx)

    oh = (e_g[:, None] == jnp.arange(N_EXP)[None, :]).astype(jnp.int32)
    incl = jnp.take_along_axis(jnp.cumsum(oh, axis=0), e_g[:, None], axis=1)[:, 0]
    survive = incl <= CAP

    surv_loc = lax.dynamic_slice(survive, (me * N_TOK,), (N_TOK,))
    e_loc = route_idx[:, 0]

    mask = (
        (e_loc[:, None] == jnp.arange(N_EXP)[None, :]) & surv_loc[:, None]
    ).astype(jnp.bfloat16)

    h_idx = jnp.arange(N_EXP) // E_LOC
    k_idx = jnp.arange(N_EXP) % E_LOC
    cols = ((me - h_idx) % N_DEV) * E_LOC + k_idx
    mask_hop = jnp.take(mask, cols, axis=1)

    out = _moe_ring(
        x.astype(jnp.bfloat16),
        mask_hop,
        expert_W.astype(jnp.bfloat16),
    )
    return out


# baseline (device time: 332309 ns/iter reference)
import jax
import jax.numpy as jnp
from jax import lax
from jax.experimental import pallas as pl
from jax.experimental.pallas import tpu as pltpu

N_DEV = 4
N_EXP = 32
E_LOC = 8
CAP = 204
N_TOK = 2048
D = 512
H = 1024


def _ring_ag_routes(route_idx):
    r16 = route_idx.reshape(16, 128)

    def body(r_ref, out_ref, comm, ssem, rsem):
        me = lax.axis_index("i")
        left = (me - 1) % N_DEV
        right = (me + 1) % N_DEV
        bar = pltpu.get_barrier_semaphore()
        for nbr in (left, right):
            pl.semaphore_signal(
                bar, inc=1, device_id=(nbr,),
                device_id_type=pl.DeviceIdType.MESH,
            )
        pl.semaphore_wait(bar, 2)

        out_ref[pl.ds(me * 16, 16), :] = r_ref[...]
        for h in range(N_DEV - 1):
            src = r_ref if h == 0 else comm.at[h - 1]
            rdma = pltpu.make_async_remote_copy(
                src_ref=src,
                dst_ref=comm.at[h],
                send_sem=ssem.at[h],
                recv_sem=rsem.at[h],
                device_id=(right,),
                device_id_type=pl.DeviceIdType.MESH,
            )
            rdma.start()
            rdma.wait()
            origin = (me - 1 - h) % N_DEV
            out_ref[pl.ds(origin * 16, 16), :] = comm[h]

    out = pl.pallas_call(
        body,
        out_shape=jax.ShapeDtypeStruct((N_DEV * 16, 128), jnp.int32),
        in_specs=[pl.BlockSpec(memory_space=pltpu.VMEM)],
        out_specs=pl.BlockSpec(memory_space=pltpu.VMEM),
        scratch_shapes=[
            pltpu.VMEM((N_DEV - 1, 16, 128), jnp.int32),
            pltpu.SemaphoreType.DMA((N_DEV - 1,)),
            pltpu.SemaphoreType.DMA((N_DEV - 1,)),
        ],
        compiler_params=pltpu.CompilerParams(collective_id=0),
    )(r16)
    return out.reshape(N_DEV * N_TOK)


def _moe_ring(x_bf, mask_bf, w_bf):

    H_CHUNK = 512

    def body(x_ref, m_ref, w_ref, out_ref, comm, ssem, rsem, credit):
        me = lax.axis_index("i")
        left = (me - 1) % N_DEV
        right = (me + 1) % N_DEV
        bar = pltpu.get_barrier_semaphore()
        for nbr in (left, right):
            pl.semaphore_signal(
                bar, inc=1, device_id=(nbr,),
                device_id_type=pl.DeviceIdType.MESH,
            )
        pl.semaphore_wait(bar, 2)

        xv = x_ref[...]
        mv = m_ref[...]
        out_ref[...] = jnp.zeros((N_TOK, H), jnp.bfloat16)

        def block(get_w, h):
            for k in range(E_LOC):
                col = h * E_LOC + k
                xm = xv * mv[:, col:col + 1]
                wk = get_w(k)
                for c in range(0, H, H_CHUNK):
                    p = jnp.dot(
                        xm, wk[:, c:c + H_CHUNK],
                        preferred_element_type=jnp.float32,
                    )
                    out_ref[:, c:c + H_CHUNK] += p.astype(jnp.bfloat16)

        r1 = pltpu.make_async_remote_copy(
            src_ref=w_ref, dst_ref=comm.at[0],
            send_sem=ssem.at[0], recv_sem=rsem.at[0],
            device_id=(right,), device_id_type=pl.DeviceIdType.MESH,
        )
        r1.start()
        block(lambda k: w_ref[k], 0)
        r1.wait()

        r2 = pltpu.make_async_remote_copy(
            src_ref=comm.at[0], dst_ref=comm.at[1],
            send_sem=ssem.at[1], recv_sem=rsem.at[1],
            device_id=(right,), device_id_type=pl.DeviceIdType.MESH,
        )
        r2.start()
        block(lambda k: comm[0, k], 1)
        r2.wait()

        pl.semaphore_signal(
            credit, inc=1, device_id=(left,),
            device_id_type=pl.DeviceIdType.MESH,
        )
        pl.semaphore_wait(credit, 1)

        r3 = pltpu.make_async_remote_copy(
            src_ref=comm.at[1], dst_ref=comm.at[0],
            send_sem=ssem.at[2], recv_sem=rsem.at[2],
            device_id=(right,), device_id_type=pl.DeviceIdType.MESH,
        )
        r3.start()
        block(lambda k: comm[1, k], 2)
        r3.wait()

        block(lambda k: comm[0, k], 3)

    return pl.pallas_call(
        body,
        out_shape=jax.ShapeDtypeStruct((N_TOK, H), jnp.bfloat16),
        in_specs=[
            pl.BlockSpec(memory_space=pltpu.VMEM),
            pl.BlockSpec(memory_space=pltpu.VMEM),
            pl.BlockSpec(memory_space=pltpu.VMEM),
        ],
        out_specs=pl.BlockSpec(memory_space=pltpu.VMEM),
        scratch_shapes=[
            pltpu.VMEM((2, E_LOC, D, H), jnp.bfloat16),
            pltpu.SemaphoreType.DMA((N_DEV - 1,)),
            pltpu.SemaphoreType.DMA((N_DEV - 1,)),
            pltpu.SemaphoreType.REGULAR,
        ],
        compiler_params=pltpu.CompilerParams(collective_id=1),
    )(x_bf, mask_bf, w_bf)


def kernel(x, router_W, route_idx, expert_W):
    del router_W

    me = lax.axis_index("i")

    e_g = _ring_ag_routes(route_idx)

    oh = (e_g[:, None] == jnp.arange(N_EXP)[None, :]).astype(jnp.int32)
    incl = jnp.take_along_axis(jnp.cumsum(oh, axis=0), e_g[:, None], axis=1)[:, 0]
    survive = incl <= CAP

    surv_loc = lax.dynamic_slice(survive, (me * N_TOK,), (N_TOK,))
    e_loc = route_idx[:, 0]

    mask = (
        (e_loc[:, None] == jnp.arange(N_EXP)[None, :]) & surv_loc[:, None]
    ).astype(jnp.bfloat16)

    h_idx = jnp.arange(N_EXP) // E_LOC
    k_idx = jnp.arange(N_EXP) % E_LOC
    cols = ((me - h_idx) % N_DEV) * E_LOC + k_idx
    mask_hop = jnp.take(mask, cols, axis=1)

    out = _moe_ring(
        x.astype(jnp.bfloat16),
        mask_hop,
        expert_W.astype(jnp.bfloat16),
    )
    return out


# device time: 234367 ns/iter; 1.4179x vs baseline; 1.4179x over previous
import jax
import jax.numpy as jnp
from jax import lax
from jax.experimental import pallas as pl
from jax.experimental.pallas import tpu as pltpu

N_DEV = 4
N_EXP = 32
E_LOC = 8
CAP = 204
N_TOK = 2048
D = 512
H = 1024
CAPP = 256
CAP_OUT = 768

_MESH = pl.DeviceIdType.MESH


def _all_barrier(me):
    bar = pltpu.get_barrier_semaphore()
    for o in range(1, N_DEV):
        pl.semaphore_signal(
            bar, inc=1, device_id=((me + o) % N_DEV,), device_id_type=_MESH,
        )
    pl.semaphore_wait(bar, N_DEV - 1)


def _ring_ag_routes(route_idx):
    r16 = route_idx.reshape(16, 128)

    def body(r_ref, out_ref, comm, ssem, rsem):
        me = lax.axis_index("i")
        right = (me + 1) % N_DEV
        _all_barrier(me)
        out_ref[pl.ds(me * 16, 16), :] = r_ref[...]
        for h in range(N_DEV - 1):
            src = r_ref if h == 0 else comm.at[h - 1]
            rdma = pltpu.make_async_remote_copy(
                src_ref=src,
                dst_ref=comm.at[h],
                send_sem=ssem.at[h],
                recv_sem=rsem.at[h],
                device_id=(right,),
                device_id_type=_MESH,
            )
            rdma.start()
            rdma.wait()
            origin = (me - 1 - h) % N_DEV
            out_ref[pl.ds(origin * 16, 16), :] = comm[h]

    out = pl.pallas_call(
        body,
        out_shape=jax.ShapeDtypeStruct((N_DEV * 16, 128), jnp.int32),
        in_specs=[pl.BlockSpec(memory_space=pltpu.VMEM)],
        out_specs=pl.BlockSpec(memory_space=pltpu.VMEM),
        scratch_shapes=[
            pltpu.VMEM((N_DEV - 1, 16, 128), jnp.int32),
            pltpu.SemaphoreType.DMA((N_DEV - 1,)),
            pltpu.SemaphoreType.DMA((N_DEV - 1,)),
        ],
        compiler_params=pltpu.CompilerParams(collective_id=0),
    )(r16)
    return out.reshape(N_DEV * N_TOK)


def _ag_x(x_bf):

    def body(x_ref, out_ref, ssem, rsem):
        me = lax.axis_index("i")
        _all_barrier(me)
        out_ref[pl.ds(me * N_TOK, N_TOK), :] = x_ref[...]
        sends = []
        for o in range(1, N_DEV):
            peer = (me + o) % N_DEV
            r = pltpu.make_async_remote_copy(
                src_ref=x_ref,
                dst_ref=out_ref.at[pl.ds(me * N_TOK, N_TOK)],
                send_sem=ssem.at[o - 1],
                recv_sem=rsem.at[o - 1],
                device_id=(peer,),
                device_id_type=_MESH,
            )
            r.start()
            sends.append(r)
        for o in range(1, N_DEV):
            src = (me - o) % N_DEV
            recv = pltpu.make_async_remote_copy(
                src_ref=x_ref,
                dst_ref=out_ref.at[pl.ds(src * N_TOK, N_TOK)],
                send_sem=ssem.at[o - 1],
                recv_sem=rsem.at[o - 1],
                device_id=(me,),
                device_id_type=_MESH,
            )
            recv.wait_recv()
        for r in sends:
            r.wait_send()

    return pl.pallas_call(
        body,
        out_shape=jax.ShapeDtypeStruct((N_DEV * N_TOK, D), jnp.bfloat16),
        in_specs=[pl.BlockSpec(memory_space=pltpu.VMEM)],
        out_specs=pl.BlockSpec(memory_space=pltpu.VMEM),
        scratch_shapes=[
            pltpu.SemaphoreType.DMA((N_DEV - 1,)),
            pltpu.SemaphoreType.DMA((N_DEV - 1,)),
        ],
        compiler_params=pltpu.CompilerParams(collective_id=1),
    )(x_bf)


def _mm(xe, w_bf):

    def body(xe_ref, w_ref, out_ref):
        for e in range(E_LOC):
            out_ref[e] = jnp.dot(
                xe_ref[e], w_ref[e], preferred_element_type=jnp.float32
            ).astype(jnp.bfloat16)

    return pl.pallas_call(
        body,
        out_shape=jax.ShapeDtypeStruct((E_LOC, CAPP, H), jnp.bfloat16),
        in_specs=[
            pl.BlockSpec(memory_space=pltpu.VMEM),
            pl.BlockSpec(memory_space=pltpu.VMEM),
        ],
        out_specs=pl.BlockSpec(memory_space=pltpu.VMEM),
    )(xe, w_bf)


def _comb(y_all):

    def body(y_ref, stk_ref, ssem, rsem):
        me = lax.axis_index("i")
        _all_barrier(me)
        stk_ref[pl.ds(me, 1)] = y_ref[pl.ds(me, 1)]
        sends = []
        for o in range(1, N_DEV):
            peer = (me + o) % N_DEV
            r = pltpu.make_async_remote_copy(
                src_ref=y_ref.at[peer],
                dst_ref=stk_ref.at[me],
                send_sem=ssem.at[o - 1],
                recv_sem=rsem.at[o - 1],
                device_id=(peer,),
                device_id_type=_MESH,
            )
            r.start()
            sends.append(r)
        for o in range(1, N_DEV):
            src = (me - o) % N_DEV
            recv = pltpu.make_async_remote_copy(
                src_ref=y_ref.at[0],
                dst_ref=stk_ref.at[src],
                send_sem=ssem.at[o - 1],
                recv_sem=rsem.at[o - 1],
                device_id=(me,),
                device_id_type=_MESH,
            )
            recv.wait_recv()
        for r in sends:
            r.wait_send()

    return pl.pallas_call(
        body,
        out_shape=jax.ShapeDtypeStruct((N_DEV, CAP_OUT, H), jnp.bfloat16),
        in_specs=[pl.BlockSpec(memory_space=pltpu.VMEM)],
        out_specs=pl.BlockSpec(memory_space=pltpu.VMEM),
        scratch_shapes=[
            pltpu.SemaphoreType.DMA((N_DEV - 1,)),
            pltpu.SemaphoreType.DMA((N_DEV - 1,)),
        ],
        compiler_params=pltpu.CompilerParams(collective_id=2),
    )(y_all)


def kernel(x, router_W, route_idx, expert_W):
    del router_W

    me = lax.axis_index("i")

    e_g = _ring_ag_routes(route_idx)

    oh = (e_g[:, None] == jnp.arange(N_EXP)[None, :]).astype(jnp.int32)
    incl = jnp.take_along_axis(jnp.cumsum(oh, axis=0), e_g[:, None], axis=1)[:, 0]
    survive = incl <= CAP
    r = incl - 1
    tok = jnp.arange(N_DEV * N_TOK, dtype=jnp.int32)
    owner = tok // N_TOK
    eo = e_g // E_LOC
    el = e_g % E_LOC

    pair = owner * N_DEV + eo
    poh = (survive[:, None] & (pair[:, None] == jnp.arange(16)[None, :])).astype(
        jnp.int32
    )
    prank = (
        jnp.take_along_axis(jnp.cumsum(poh, axis=0), pair[:, None], axis=1)[:, 0] - 1
    )

    xg = _ag_x(x.astype(jnp.bfloat16))

    mine = survive & (eo == me)
    gi_t = jnp.where(mine, el * CAPP + r, E_LOC * CAPP)
    gi = (
        jnp.zeros(E_LOC * CAPP, jnp.int32).at[gi_t].set(tok, mode="drop")
    )
    xe = xg[gi].reshape(E_LOC, CAPP, D)

    yg = _mm(xe, expert_W.astype(jnp.bfloat16))

    tgt = jnp.where(mine, owner * CAP_OUT + prank, N_DEV * CAP_OUT)
    pidx = (
        jnp.zeros(N_DEV * CAP_OUT, jnp.int32).at[tgt].set(el * CAPP + r, mode="drop")
    )
    y_all = yg.reshape(E_LOC * CAPP, H)[pidx].reshape(N_DEV, CAP_OUT, H)

    stk = _comb(y_all)

    surv_loc = lax.dynamic_slice(survive, (me * N_TOK,), (N_TOK,))
    eo_loc = lax.dynamic_slice(eo, (me * N_TOK,), (N_TOK,))
    prank_loc = lax.dynamic_slice(prank, (me * N_TOK,), (N_TOK,))
    rowidx = jnp.where(surv_loc, eo_loc * CAP_OUT + prank_loc, 0)
    out = stk.reshape(N_DEV * CAP_OUT, H)[rowidx] * surv_loc[:, None].astype(
        jnp.bfloat16
    )
    return out
